# baseline (speedup 1.0000x reference)
import jax
import jax.numpy as jnp
from jax.experimental import pallas as pl

DIM = 64
N_LAYER = 2
B = 4096
L = 50
BB = 64


def _attn(h, t, q):
    sims = jnp.sum(h * q[:, None, :], axis=-1)
    w = jax.nn.softmax(sims, axis=-1)
    return jnp.sum(w[..., None] * t, axis=1)


def _tower_kernel(io_ref, ih0, it0, ih1, it1, uh0, ut0, uh1, ut1, out_ref):
    q = io_ref[...]
    e_v = q + _attn(ih0[...], it0[...], q) + _attn(ih1[...], it1[...], q)
    u0 = uh0[...]
    qu = jnp.mean(u0, axis=1)
    e_u = qu + _attn(u0, ut0[...], qu) + _attn(uh1[...], ut1[...], qu)
    out_ref[...] = jax.nn.sigmoid(jnp.sum(e_u * e_v, axis=-1))[None, None, :]


def kernel(items, user_triple_set, item_triple_set, entity_emb, relation_emb):
    del relation_emb
    io = jnp.take(entity_emb, items, axis=0)
    ih = jnp.take(entity_emb, item_triple_set[0].reshape(-1), axis=0)
    it = jnp.take(entity_emb, item_triple_set[2].reshape(-1), axis=0)
    uh = jnp.take(entity_emb, user_triple_set[0].reshape(-1), axis=0)
    ut = jnp.take(entity_emb, user_triple_set[2].reshape(-1), axis=0)
    ih = ih.reshape(N_LAYER, B, L, DIM)
    it = it.reshape(N_LAYER, B, L, DIM)
    uh = uh.reshape(N_LAYER, B, L, DIM)
    ut = ut.reshape(N_LAYER, B, L, DIM)

    bld = pl.BlockSpec((BB, L, DIM), lambda i: (i, 0, 0))
    bd = pl.BlockSpec((BB, DIM), lambda i: (i, 0))
    scores = pl.pallas_call(
        _tower_kernel,
        grid=(B // BB,),
        in_specs=[bd, bld, bld, bld, bld, bld, bld, bld, bld],
        out_specs=pl.BlockSpec((1, 1, BB), lambda i: (i, 0, 0)),
        out_shape=jax.ShapeDtypeStruct((B // BB, 1, BB), jnp.float32),
    )(io, ih[0], it[0], ih[1], it[1], uh[0], ut[0], uh[1], ut[1])
    return scores.reshape(B)
